# baseline (speedup 1.0000x reference)
import functools

import jax
import jax.numpy as jnp
from jax import lax
from jax.experimental import pallas as pl
from jax.experimental.pallas import tpu as pltpu
from jax.experimental.pallas import tpu_sc as plsc

HIDDEN = 128
N_NODES = 10000
E_PAD = 327680
ACC_ROWS = 10240
EB = 2048
W = 128
N_SUBCORES = 16
NW = 2 * N_SUBCORES
NBUF = 2

C = 4
CHUNK_BLOCKS = (8, 24, 64, 64)


def _sc_mesh():
    return plsc.VectorSubcoreMesh(core_axis_name="c", subcore_axis_name="s")


def _gather(table, idx):
    cw = idx.shape[0]
    wsteps = cw // NW

    @functools.partial(
        pl.kernel,
        out_type=jax.ShapeDtypeStruct((cw * W, HIDDEN), jnp.float32),
        mesh=_sc_mesh(),
        scratch_types=[
            pltpu.VMEM_SHARED((ACC_ROWS, HIDDEN), jnp.float32),
            pltpu.VMEM((wsteps, W), jnp.int32),
            pltpu.VMEM((NBUF, W, HIDDEN), jnp.float32),
            pltpu.SemaphoreType.DMA((NBUF,)),
            pltpu.SemaphoreType.DMA((NBUF,)),
        ],
    )
    def kern(table_hbm, idx_hbm, out_hbm, table_s, idx_v, bufs, gsem, osem):
        s = lax.axis_index("s")
        wid = lax.axis_index("c") * N_SUBCORES + s
        stripe = ACC_ROWS // N_SUBCORES
        r0 = s * stripe
        pltpu.sync_copy(table_hbm.at[pl.ds(r0, stripe)],
                        table_s.at[pl.ds(r0, stripe)])
        pltpu.sync_copy(idx_hbm.at[pl.ds(wid * wsteps, wsteps)], idx_v)
        plsc.subcore_barrier()

        def out_slot(w):
            return out_hbm.at[pl.ds((wid * wsteps + w) * W, W)]

        for b in range(NBUF):
            pltpu.async_copy(table_s.at[idx_v.at[b]], bufs.at[b], gsem.at[b])

        @pl.loop(0, wsteps // NBUF)
        def _(k):
            for b in range(NBUF):
                w = k * NBUF + b
                pltpu.make_async_copy(
                    table_s.at[idx_v.at[w]], bufs.at[b], gsem.at[b]).wait()
                pltpu.async_copy(bufs.at[b], out_slot(w), osem.at[b])

                @pl.when(k < wsteps // NBUF - 1)
                def _():
                    pltpu.make_async_copy(
                        bufs.at[b], out_slot(w), osem.at[b]).wait()
                    pltpu.async_copy(table_s.at[idx_v.at[w + NBUF]],
                                     bufs.at[b], gsem.at[b])

        for b in range(NBUF):
            pltpu.make_async_copy(
                bufs.at[b], out_slot(wsteps - NBUF + b), osem.at[b]).wait()

    return kern(table, idx)


def _scatter_add(msgs, ridx, zeros, sww):
    @functools.partial(
        pl.kernel,
        out_type=jax.ShapeDtypeStruct((2, ACC_ROWS, HIDDEN), jnp.float32),
        mesh=_sc_mesh(),
        scratch_types=[
            pltpu.VMEM_SHARED((ACC_ROWS, HIDDEN), jnp.float32),
            pltpu.VMEM((sww, W), jnp.int32),
            pltpu.VMEM((2, W, HIDDEN), jnp.float32),
            pltpu.SemaphoreType.DMA((2,)),
        ],
    )
    def kern(m, ridx_hbm, zeros_hbm, part_hbm, acc, idx_v, mbuf, msem):
        c = lax.axis_index("c")
        s = lax.axis_index("s")
        wid = c * N_SUBCORES + s
        stripe = ACC_ROWS // N_SUBCORES
        r0 = s * stripe
        pltpu.sync_copy(zeros_hbm.at[pl.ds(r0, stripe)],
                        acc.at[pl.ds(r0, stripe)])
        pltpu.sync_copy(ridx_hbm.at[wid], idx_v)
        plsc.subcore_barrier()

        def mslot(t):
            return m.at[pl.ds((wid * sww + t) * W, W)]

        for b in range(2):
            pltpu.async_copy(mslot(b), mbuf.at[b], msem.at[b])
        for t in range(sww):
            b = t % 2
            pltpu.make_async_copy(mslot(t), mbuf.at[b], msem.at[b]).wait()
            pltpu.sync_copy(mbuf.at[b], acc.at[idx_v.at[t]], add=True)
            if t + 2 < sww:
                pltpu.async_copy(mslot(t + 2), mbuf.at[b], msem.at[b])

        plsc.subcore_barrier()
        pltpu.sync_copy(acc.at[pl.ds(r0, stripe)],
                        part_hbm.at[c, pl.ds(r0, stripe)])

    return kern(msgs, ridx, zeros)


def _mlp_body(gs_ref, gr_ref, el_ref, w1s_ref, w1r_ref, w1e_ref, b1_ref,
              w2_ref, b2_ref, o_ref):
    dn_t = (((0,), (1,)), ((), ()))
    x = lax.dot_general(w1s_ref[...], gs_ref[...].astype(jnp.bfloat16), dn_t,
                        preferred_element_type=jnp.float32)
    x += lax.dot_general(w1r_ref[...], gr_ref[...].astype(jnp.bfloat16), dn_t,
                         preferred_element_type=jnp.float32)
    el = el_ref[0]
    x += lax.dot_general(w1e_ref[...], el, (((0,), (0,)), ((), ())),
                         preferred_element_type=jnp.float32)
    x += b1_ref[...]
    hmid = (x * jax.nn.sigmoid(x)).astype(jnp.bfloat16)
    m = lax.dot_general(hmid, w2_ref[...], (((0,), (0,)), ((), ())),
                        preferred_element_type=jnp.float32)
    o_ref[...] = m + b2_ref[...]


def _mlp(gathered, el3, w1s, w1r, w1e, b1c, w2, b2r):
    nbc = gathered.shape[0] // (2 * EB)
    return pl.pallas_call(
        _mlp_body,
        grid=(nbc,),
        in_specs=[
            pl.BlockSpec((EB, HIDDEN), lambda i: (i, 0)),
            pl.BlockSpec((EB, HIDDEN), lambda i: (i + nbc, 0)),
            pl.BlockSpec((1, 1, EB), lambda i: (i, 0, 0)),
            pl.BlockSpec((HIDDEN, HIDDEN), lambda i: (0, 0)),
            pl.BlockSpec((HIDDEN, HIDDEN), lambda i: (0, 0)),
            pl.BlockSpec((1, HIDDEN), lambda i: (0, 0)),
            pl.BlockSpec((HIDDEN, 1), lambda i: (0, 0)),
            pl.BlockSpec((HIDDEN, HIDDEN), lambda i: (0, 0)),
            pl.BlockSpec((1, HIDDEN), lambda i: (0, 0)),
        ],
        out_specs=pl.BlockSpec((EB, HIDDEN), lambda i: (i, 0)),
        out_shape=jax.ShapeDtypeStruct((nbc * EB, HIDDEN), jnp.float32),
    )(gathered, gathered, el3, w1s, w1r, w1e, b1c, w2, b2r)


def _final_body(h_ref, p0_ref, p1_ref, p2_ref, p3_ref, o_ref):
    agg = p0_ref[0] + p0_ref[1] + p1_ref[0] + p1_ref[1]
    agg += p2_ref[0] + p2_ref[1] + p3_ref[0] + p3_ref[1]
    o_ref[:, :HIDDEN] = h_ref[:, :HIDDEN] + agg
    o_ref[:, HIDDEN:] = h_ref[:, HIDDEN:]


def _finalize(h, parts):
    n, f = h.shape
    rb = 1000
    pspec = pl.BlockSpec((2, rb, HIDDEN), lambda i: (0, i, 0))
    return pl.pallas_call(
        _final_body,
        grid=(n // rb,),
        in_specs=[pl.BlockSpec((rb, f), lambda i: (i, 0))] + [pspec] * C,
        out_specs=pl.BlockSpec((rb, f), lambda i: (i, 0)),
        out_shape=jax.ShapeDtypeStruct((n, f), jnp.float32),
    )(h, *parts)


def kernel(h, edge_index, edge_len, W1, b1, W2, b2):
    scalars = jnp.pad(h[:, :HIDDEN], ((0, ACC_ROWS - N_NODES), (0, 0)))
    sender = edge_index[0].astype(jnp.int32)
    receiver = edge_index[1].astype(jnp.int32)
    e = sender.shape[0]
    pad = E_PAD - e
    sender_p = jnp.pad(sender, (0, pad))
    receiver_p = jnp.pad(receiver, (0, pad), constant_values=N_NODES)
    el_p = jnp.pad(edge_len.astype(jnp.float32), (0, pad))

    w1s = W1[:HIDDEN].astype(jnp.bfloat16)
    w1r = W1[HIDDEN:2 * HIDDEN].astype(jnp.bfloat16)
    w1e = W1[2 * HIDDEN:]
    b1c = b1.reshape(HIDDEN, 1)
    w2 = W2.astype(jnp.bfloat16)
    b2r = b2.reshape(1, HIDDEN)

    zeros = jnp.zeros((ACC_ROWS, HIDDEN), jnp.float32)
    parts = []
    off = 0
    for nb in CHUNK_BLOCKS:
        ec = nb * EB
        sl = slice(off, off + ec)
        off += ec
        idx_c = jnp.concatenate(
            [sender_p[sl], receiver_p[sl]]).reshape(2 * ec // W, W)
        gathered = _gather(scalars, idx_c)
        msgs = _mlp(gathered, el_p[sl].reshape(nb, 1, EB),
                    w1s, w1r, w1e, b1c, w2, b2r)
        sww = ec // W // NW
        parts.append(
            _scatter_add(msgs, receiver_p[sl].reshape(NW, sww, W), zeros, sww))
    return _finalize(h, parts)


# speedup vs baseline: 1.0424x; 1.0424x over previous
import functools

import jax
import jax.numpy as jnp
from jax import lax
from jax.experimental import pallas as pl
from jax.experimental.pallas import tpu as pltpu
from jax.experimental.pallas import tpu_sc as plsc

HIDDEN = 128
N_NODES = 10000
E_PAD = 327680
ACC_ROWS = 10240
EB = 2048
W = 128
N_SUBCORES = 16
NW = 2 * N_SUBCORES
NBUF = 2

C = 4
CHUNK_BLOCKS = (40, 40, 40, 40)


def _sc_mesh():
    return plsc.VectorSubcoreMesh(core_axis_name="c", subcore_axis_name="s")


def _gather(table, idx):
    cw = idx.shape[0]
    wsteps = cw // NW

    @functools.partial(
        pl.kernel,
        out_type=jax.ShapeDtypeStruct((cw * W, HIDDEN), jnp.float32),
        mesh=_sc_mesh(),
        scratch_types=[
            pltpu.VMEM_SHARED((ACC_ROWS, HIDDEN), jnp.float32),
            pltpu.VMEM((wsteps, W), jnp.int32),
            pltpu.VMEM((NBUF, W, HIDDEN), jnp.float32),
            pltpu.SemaphoreType.DMA((NBUF,)),
            pltpu.SemaphoreType.DMA((NBUF,)),
        ],
    )
    def kern(table_hbm, idx_hbm, out_hbm, table_s, idx_v, bufs, gsem, osem):
        s = lax.axis_index("s")
        wid = lax.axis_index("c") * N_SUBCORES + s
        stripe = ACC_ROWS // N_SUBCORES
        r0 = s * stripe
        pltpu.sync_copy(table_hbm.at[pl.ds(r0, stripe)],
                        table_s.at[pl.ds(r0, stripe)])
        pltpu.sync_copy(idx_hbm.at[pl.ds(wid * wsteps, wsteps)], idx_v)
        plsc.subcore_barrier()

        def out_slot(w):
            return out_hbm.at[pl.ds((wid * wsteps + w) * W, W)]

        for b in range(NBUF):
            pltpu.async_copy(table_s.at[idx_v.at[b]], bufs.at[b], gsem.at[b])

        @pl.loop(0, wsteps // NBUF)
        def _(k):
            for b in range(NBUF):
                w = k * NBUF + b
                pltpu.make_async_copy(
                    table_s.at[idx_v.at[w]], bufs.at[b], gsem.at[b]).wait()
                pltpu.async_copy(bufs.at[b], out_slot(w), osem.at[b])

                @pl.when(k < wsteps // NBUF - 1)
                def _():
                    pltpu.make_async_copy(
                        bufs.at[b], out_slot(w), osem.at[b]).wait()
                    pltpu.async_copy(table_s.at[idx_v.at[w + NBUF]],
                                     bufs.at[b], gsem.at[b])

        for b in range(NBUF):
            pltpu.make_async_copy(
                bufs.at[b], out_slot(wsteps - NBUF + b), osem.at[b]).wait()

    return kern(table, idx)


def _scatter_add(msgs, ridx, zeros, sww):
    @functools.partial(
        pl.kernel,
        out_type=jax.ShapeDtypeStruct((2, ACC_ROWS, HIDDEN), jnp.float32),
        mesh=_sc_mesh(),
        scratch_types=[
            pltpu.VMEM_SHARED((ACC_ROWS, HIDDEN), jnp.float32),
            pltpu.VMEM((sww, W), jnp.int32),
            pltpu.VMEM((2, W, HIDDEN), jnp.float32),
            pltpu.SemaphoreType.DMA((2,)),
        ],
    )
    def kern(m, ridx_hbm, zeros_hbm, part_hbm, acc, idx_v, mbuf, msem):
        c = lax.axis_index("c")
        s = lax.axis_index("s")
        wid = c * N_SUBCORES + s
        stripe = ACC_ROWS // N_SUBCORES
        r0 = s * stripe
        pltpu.sync_copy(zeros_hbm.at[pl.ds(r0, stripe)],
                        acc.at[pl.ds(r0, stripe)])
        pltpu.sync_copy(ridx_hbm.at[wid], idx_v)
        plsc.subcore_barrier()

        def mslot(t):
            return m.at[pl.ds((wid * sww + t) * W, W)]

        for b in range(2):
            pltpu.async_copy(mslot(b), mbuf.at[b], msem.at[b])
        for t in range(sww):
            b = t % 2
            pltpu.make_async_copy(mslot(t), mbuf.at[b], msem.at[b]).wait()
            pltpu.sync_copy(mbuf.at[b], acc.at[idx_v.at[t]], add=True)
            if t + 2 < sww:
                pltpu.async_copy(mslot(t + 2), mbuf.at[b], msem.at[b])

        plsc.subcore_barrier()
        pltpu.sync_copy(acc.at[pl.ds(r0, stripe)],
                        part_hbm.at[c, pl.ds(r0, stripe)])

    return kern(msgs, ridx, zeros)


def _mlp_body(gs_ref, gr_ref, el_ref, w1s_ref, w1r_ref, w1e_ref, b1_ref,
              w2_ref, b2_ref, o_ref):
    dn_t = (((0,), (1,)), ((), ()))
    x = lax.dot_general(w1s_ref[...], gs_ref[...].astype(jnp.bfloat16), dn_t,
                        preferred_element_type=jnp.float32)
    x += lax.dot_general(w1r_ref[...], gr_ref[...].astype(jnp.bfloat16), dn_t,
                         preferred_element_type=jnp.float32)
    el = el_ref[0]
    x += lax.dot_general(w1e_ref[...], el, (((0,), (0,)), ((), ())),
                         preferred_element_type=jnp.float32)
    x += b1_ref[...]
    hmid = (x * jax.nn.sigmoid(x)).astype(jnp.bfloat16)
    m = lax.dot_general(hmid, w2_ref[...], (((0,), (0,)), ((), ())),
                        preferred_element_type=jnp.float32)
    o_ref[...] = m + b2_ref[...]


def _mlp(gathered, el3, w1s, w1r, w1e, b1c, w2, b2r):
    nbc = gathered.shape[0] // (2 * EB)
    return pl.pallas_call(
        _mlp_body,
        grid=(nbc,),
        in_specs=[
            pl.BlockSpec((EB, HIDDEN), lambda i: (i, 0)),
            pl.BlockSpec((EB, HIDDEN), lambda i: (i + nbc, 0)),
            pl.BlockSpec((1, 1, EB), lambda i: (i, 0, 0)),
            pl.BlockSpec((HIDDEN, HIDDEN), lambda i: (0, 0)),
            pl.BlockSpec((HIDDEN, HIDDEN), lambda i: (0, 0)),
            pl.BlockSpec((1, HIDDEN), lambda i: (0, 0)),
            pl.BlockSpec((HIDDEN, 1), lambda i: (0, 0)),
            pl.BlockSpec((HIDDEN, HIDDEN), lambda i: (0, 0)),
            pl.BlockSpec((1, HIDDEN), lambda i: (0, 0)),
        ],
        out_specs=pl.BlockSpec((EB, HIDDEN), lambda i: (i, 0)),
        out_shape=jax.ShapeDtypeStruct((nbc * EB, HIDDEN), jnp.float32),
    )(gathered, gathered, el3, w1s, w1r, w1e, b1c, w2, b2r)


def _final_body(h_ref, p0_ref, p1_ref, p2_ref, p3_ref, o_ref):
    agg = p0_ref[0] + p0_ref[1] + p1_ref[0] + p1_ref[1]
    agg += p2_ref[0] + p2_ref[1] + p3_ref[0] + p3_ref[1]
    o_ref[:, :HIDDEN] = h_ref[:, :HIDDEN] + agg
    o_ref[:, HIDDEN:] = h_ref[:, HIDDEN:]


def _finalize(h, parts):
    n, f = h.shape
    rb = 1000
    pspec = pl.BlockSpec((2, rb, HIDDEN), lambda i: (0, i, 0))
    return pl.pallas_call(
        _final_body,
        grid=(n // rb,),
        in_specs=[pl.BlockSpec((rb, f), lambda i: (i, 0))] + [pspec] * C,
        out_specs=pl.BlockSpec((rb, f), lambda i: (i, 0)),
        out_shape=jax.ShapeDtypeStruct((n, f), jnp.float32),
    )(h, *parts)


def kernel(h, edge_index, edge_len, W1, b1, W2, b2):
    scalars = jnp.pad(h[:, :HIDDEN], ((0, ACC_ROWS - N_NODES), (0, 0)))
    sender = edge_index[0].astype(jnp.int32)
    receiver = edge_index[1].astype(jnp.int32)
    e = sender.shape[0]
    pad = E_PAD - e
    sender_p = jnp.pad(sender, (0, pad))
    receiver_p = jnp.pad(receiver, (0, pad), constant_values=N_NODES)
    el_p = jnp.pad(edge_len.astype(jnp.float32), (0, pad))

    w1s = W1[:HIDDEN].astype(jnp.bfloat16)
    w1r = W1[HIDDEN:2 * HIDDEN].astype(jnp.bfloat16)
    w1e = W1[2 * HIDDEN:]
    b1c = b1.reshape(HIDDEN, 1)
    w2 = W2.astype(jnp.bfloat16)
    b2r = b2.reshape(1, HIDDEN)

    zeros = jnp.zeros((ACC_ROWS, HIDDEN), jnp.float32)
    parts = []
    off = 0
    for nb in CHUNK_BLOCKS:
        ec = nb * EB
        sl = slice(off, off + ec)
        off += ec
        idx_c = jnp.concatenate(
            [sender_p[sl], receiver_p[sl]]).reshape(2 * ec // W, W)
        gathered = _gather(scalars, idx_c)
        msgs = _mlp(gathered, el_p[sl].reshape(nb, 1, EB),
                    w1s, w1r, w1e, b1c, w2, b2r)
        sww = ec // W // NW
        parts.append(
            _scatter_add(msgs, receiver_p[sl].reshape(NW, sww, W), zeros, sww))
    return _finalize(h, parts)


# speedup vs baseline: 1.0449x; 1.0024x over previous
import functools

import jax
import jax.numpy as jnp
from jax import lax
from jax.experimental import pallas as pl
from jax.experimental.pallas import tpu as pltpu
from jax.experimental.pallas import tpu_sc as plsc

HIDDEN = 128
N_NODES = 10000
E_PAD = 327680
ACC_ROWS = 10240
EB = 2048
W = 128
N_SUBCORES = 16
NW = 2 * N_SUBCORES
NBUF = 2

C = 4
CHUNK_BLOCKS = (24, 48, 64, 24)


def _sc_mesh():
    return plsc.VectorSubcoreMesh(core_axis_name="c", subcore_axis_name="s")


def _gather(table, idx):
    cw = idx.shape[0]
    wsteps = cw // NW

    @functools.partial(
        pl.kernel,
        out_type=jax.ShapeDtypeStruct((cw * W, HIDDEN), jnp.float32),
        mesh=_sc_mesh(),
        scratch_types=[
            pltpu.VMEM_SHARED((ACC_ROWS, HIDDEN), jnp.float32),
            pltpu.VMEM((wsteps, W), jnp.int32),
            pltpu.VMEM((NBUF, W, HIDDEN), jnp.float32),
            pltpu.SemaphoreType.DMA((NBUF,)),
            pltpu.SemaphoreType.DMA((NBUF,)),
        ],
    )
    def kern(table_hbm, idx_hbm, out_hbm, table_s, idx_v, bufs, gsem, osem):
        s = lax.axis_index("s")
        wid = lax.axis_index("c") * N_SUBCORES + s
        stripe = ACC_ROWS // N_SUBCORES
        r0 = s * stripe
        pltpu.sync_copy(table_hbm.at[pl.ds(r0, stripe)],
                        table_s.at[pl.ds(r0, stripe)])
        pltpu.sync_copy(idx_hbm.at[pl.ds(wid * wsteps, wsteps)], idx_v)
        plsc.subcore_barrier()

        def out_slot(w):
            return out_hbm.at[pl.ds((wid * wsteps + w) * W, W)]

        for b in range(NBUF):
            pltpu.async_copy(table_s.at[idx_v.at[b]], bufs.at[b], gsem.at[b])

        @pl.loop(0, wsteps // NBUF)
        def _(k):
            for b in range(NBUF):
                w = k * NBUF + b
                pltpu.make_async_copy(
                    table_s.at[idx_v.at[w]], bufs.at[b], gsem.at[b]).wait()
                pltpu.async_copy(bufs.at[b], out_slot(w), osem.at[b])

                @pl.when(k < wsteps // NBUF - 1)
                def _():
                    pltpu.make_async_copy(
                        bufs.at[b], out_slot(w), osem.at[b]).wait()
                    pltpu.async_copy(table_s.at[idx_v.at[w + NBUF]],
                                     bufs.at[b], gsem.at[b])

        for b in range(NBUF):
            pltpu.make_async_copy(
                bufs.at[b], out_slot(wsteps - NBUF + b), osem.at[b]).wait()

    return kern(table, idx)


def _scatter_add(msgs, ridx, zeros, sww):
    @functools.partial(
        pl.kernel,
        out_type=jax.ShapeDtypeStruct((2, ACC_ROWS, HIDDEN), jnp.float32),
        mesh=_sc_mesh(),
        scratch_types=[
            pltpu.VMEM_SHARED((ACC_ROWS, HIDDEN), jnp.float32),
            pltpu.VMEM((sww, W), jnp.int32),
            pltpu.VMEM((2, W, HIDDEN), jnp.float32),
            pltpu.SemaphoreType.DMA((2,)),
        ],
    )
    def kern(m, ridx_hbm, zeros_hbm, part_hbm, acc, idx_v, mbuf, msem):
        c = lax.axis_index("c")
        s = lax.axis_index("s")
        wid = c * N_SUBCORES + s
        stripe = ACC_ROWS // N_SUBCORES
        r0 = s * stripe
        pltpu.sync_copy(zeros_hbm.at[pl.ds(r0, stripe)],
                        acc.at[pl.ds(r0, stripe)])
        pltpu.sync_copy(ridx_hbm.at[wid], idx_v)
        plsc.subcore_barrier()

        def mslot(t):
            return m.at[pl.ds((wid * sww + t) * W, W)]

        for b in range(2):
            pltpu.async_copy(mslot(b), mbuf.at[b], msem.at[b])
        for t in range(sww):
            b = t % 2
            pltpu.make_async_copy(mslot(t), mbuf.at[b], msem.at[b]).wait()
            pltpu.sync_copy(mbuf.at[b], acc.at[idx_v.at[t]], add=True)
            if t + 2 < sww:
                pltpu.async_copy(mslot(t + 2), mbuf.at[b], msem.at[b])

        plsc.subcore_barrier()
        pltpu.sync_copy(acc.at[pl.ds(r0, stripe)],
                        part_hbm.at[c, pl.ds(r0, stripe)])

    return kern(msgs, ridx, zeros)


def _mlp_body(gs_ref, gr_ref, el_ref, w1s_ref, w1r_ref, w1e_ref, b1_ref,
              w2_ref, b2_ref, o_ref):
    dn_t = (((0,), (1,)), ((), ()))
    x = lax.dot_general(w1s_ref[...], gs_ref[...].astype(jnp.bfloat16), dn_t,
                        preferred_element_type=jnp.float32)
    x += lax.dot_general(w1r_ref[...], gr_ref[...].astype(jnp.bfloat16), dn_t,
                         preferred_element_type=jnp.float32)
    el = el_ref[0]
    x += lax.dot_general(w1e_ref[...], el, (((0,), (0,)), ((), ())),
                         preferred_element_type=jnp.float32)
    x += b1_ref[...]
    hmid = (x * jax.nn.sigmoid(x)).astype(jnp.bfloat16)
    m = lax.dot_general(hmid, w2_ref[...], (((0,), (0,)), ((), ())),
                        preferred_element_type=jnp.float32)
    o_ref[...] = m + b2_ref[...]


def _mlp(gathered, el3, w1s, w1r, w1e, b1c, w2, b2r):
    nbc = gathered.shape[0] // (2 * EB)
    return pl.pallas_call(
        _mlp_body,
        grid=(nbc,),
        in_specs=[
            pl.BlockSpec((EB, HIDDEN), lambda i: (i, 0)),
            pl.BlockSpec((EB, HIDDEN), lambda i: (i + nbc, 0)),
            pl.BlockSpec((1, 1, EB), lambda i: (i, 0, 0)),
            pl.BlockSpec((HIDDEN, HIDDEN), lambda i: (0, 0)),
            pl.BlockSpec((HIDDEN, HIDDEN), lambda i: (0, 0)),
            pl.BlockSpec((1, HIDDEN), lambda i: (0, 0)),
            pl.BlockSpec((HIDDEN, 1), lambda i: (0, 0)),
            pl.BlockSpec((HIDDEN, HIDDEN), lambda i: (0, 0)),
            pl.BlockSpec((1, HIDDEN), lambda i: (0, 0)),
        ],
        out_specs=pl.BlockSpec((EB, HIDDEN), lambda i: (i, 0)),
        out_shape=jax.ShapeDtypeStruct((nbc * EB, HIDDEN), jnp.float32),
    )(gathered, gathered, el3, w1s, w1r, w1e, b1c, w2, b2r)


def _final_body(h_ref, p0_ref, p1_ref, p2_ref, p3_ref, o_ref):
    agg = p0_ref[0] + p0_ref[1] + p1_ref[0] + p1_ref[1]
    agg += p2_ref[0] + p2_ref[1] + p3_ref[0] + p3_ref[1]
    o_ref[:, :HIDDEN] = h_ref[:, :HIDDEN] + agg
    o_ref[:, HIDDEN:] = h_ref[:, HIDDEN:]


def _finalize(h, parts):
    n, f = h.shape
    rb = 1000
    pspec = pl.BlockSpec((2, rb, HIDDEN), lambda i: (0, i, 0))
    return pl.pallas_call(
        _final_body,
        grid=(n // rb,),
        in_specs=[pl.BlockSpec((rb, f), lambda i: (i, 0))] + [pspec] * C,
        out_specs=pl.BlockSpec((rb, f), lambda i: (i, 0)),
        out_shape=jax.ShapeDtypeStruct((n, f), jnp.float32),
    )(h, *parts)


def kernel(h, edge_index, edge_len, W1, b1, W2, b2):
    scalars = jnp.pad(h[:, :HIDDEN], ((0, ACC_ROWS - N_NODES), (0, 0)))
    sender = edge_index[0].astype(jnp.int32)
    receiver = edge_index[1].astype(jnp.int32)
    e = sender.shape[0]
    pad = E_PAD - e
    sender_p = jnp.pad(sender, (0, pad))
    receiver_p = jnp.pad(receiver, (0, pad), constant_values=N_NODES)
    el_p = jnp.pad(edge_len.astype(jnp.float32), (0, pad))

    w1s = W1[:HIDDEN].astype(jnp.bfloat16)
    w1r = W1[HIDDEN:2 * HIDDEN].astype(jnp.bfloat16)
    w1e = W1[2 * HIDDEN:]
    b1c = b1.reshape(HIDDEN, 1)
    w2 = W2.astype(jnp.bfloat16)
    b2r = b2.reshape(1, HIDDEN)

    zeros = jnp.zeros((ACC_ROWS, HIDDEN), jnp.float32)
    parts = []
    off = 0
    for nb in CHUNK_BLOCKS:
        ec = nb * EB
        sl = slice(off, off + ec)
        off += ec
        idx_c = jnp.concatenate(
            [sender_p[sl], receiver_p[sl]]).reshape(2 * ec // W, W)
        gathered = _gather(scalars, idx_c)
        msgs = _mlp(gathered, el_p[sl].reshape(nb, 1, EB),
                    w1s, w1r, w1e, b1c, w2, b2r)
        sww = ec // W // NW
        parts.append(
            _scatter_add(msgs, receiver_p[sl].reshape(NW, sww, W), zeros, sww))
    return _finalize(h, parts)


# speedup vs baseline: 1.0458x; 1.0009x over previous
import functools

import jax
import jax.numpy as jnp
from jax import lax
from jax.experimental import pallas as pl
from jax.experimental.pallas import tpu as pltpu
from jax.experimental.pallas import tpu_sc as plsc

HIDDEN = 128
N_NODES = 10000
E_PAD = 327680
ACC_ROWS = 10240
EB = 2048
W = 128
N_SUBCORES = 16
NW = 2 * N_SUBCORES
NBUF = 2

CHUNK_BLOCKS = (32, 64, 64)


def _sc_mesh():
    return plsc.VectorSubcoreMesh(core_axis_name="c", subcore_axis_name="s")


def _gather(table, idx):
    cw = idx.shape[0]
    wsteps = cw // NW

    @functools.partial(
        pl.kernel,
        out_type=jax.ShapeDtypeStruct((cw * W, HIDDEN), jnp.float32),
        mesh=_sc_mesh(),
        scratch_types=[
            pltpu.VMEM_SHARED((ACC_ROWS, HIDDEN), jnp.float32),
            pltpu.VMEM((wsteps, W), jnp.int32),
            pltpu.VMEM((NBUF, W, HIDDEN), jnp.float32),
            pltpu.SemaphoreType.DMA((NBUF,)),
            pltpu.SemaphoreType.DMA((NBUF,)),
        ],
    )
    def kern(table_hbm, idx_hbm, out_hbm, table_s, idx_v, bufs, gsem, osem):
        s = lax.axis_index("s")
        wid = lax.axis_index("c") * N_SUBCORES + s
        stripe = ACC_ROWS // N_SUBCORES
        r0 = s * stripe
        pltpu.sync_copy(table_hbm.at[pl.ds(r0, stripe)],
                        table_s.at[pl.ds(r0, stripe)])
        pltpu.sync_copy(idx_hbm.at[pl.ds(wid * wsteps, wsteps)], idx_v)
        plsc.subcore_barrier()

        def out_slot(w):
            return out_hbm.at[pl.ds((wid * wsteps + w) * W, W)]

        for b in range(NBUF):
            pltpu.async_copy(table_s.at[idx_v.at[b]], bufs.at[b], gsem.at[b])

        @pl.loop(0, wsteps // NBUF)
        def _(k):
            for b in range(NBUF):
                w = k * NBUF + b
                pltpu.make_async_copy(
                    table_s.at[idx_v.at[w]], bufs.at[b], gsem.at[b]).wait()
                pltpu.async_copy(bufs.at[b], out_slot(w), osem.at[b])

                @pl.when(k < wsteps // NBUF - 1)
                def _():
                    pltpu.make_async_copy(
                        bufs.at[b], out_slot(w), osem.at[b]).wait()
                    pltpu.async_copy(table_s.at[idx_v.at[w + NBUF]],
                                     bufs.at[b], gsem.at[b])

        for b in range(NBUF):
            pltpu.make_async_copy(
                bufs.at[b], out_slot(wsteps - NBUF + b), osem.at[b]).wait()

    return kern(table, idx)


def _scatter_add(msgs, ridx, zeros, sww):
    @functools.partial(
        pl.kernel,
        out_type=jax.ShapeDtypeStruct((2, ACC_ROWS, HIDDEN), jnp.float32),
        mesh=_sc_mesh(),
        scratch_types=[
            pltpu.VMEM_SHARED((ACC_ROWS, HIDDEN), jnp.float32),
            pltpu.VMEM((sww, W), jnp.int32),
            pltpu.VMEM((2, W, HIDDEN), jnp.float32),
            pltpu.SemaphoreType.DMA((2,)),
        ],
    )
    def kern(m, ridx_hbm, zeros_hbm, part_hbm, acc, idx_v, mbuf, msem):
        c = lax.axis_index("c")
        s = lax.axis_index("s")
        wid = c * N_SUBCORES + s
        stripe = ACC_ROWS // N_SUBCORES
        r0 = s * stripe
        pltpu.sync_copy(zeros_hbm.at[pl.ds(r0, stripe)],
                        acc.at[pl.ds(r0, stripe)])
        pltpu.sync_copy(ridx_hbm.at[wid], idx_v)
        plsc.subcore_barrier()

        def mslot(t):
            return m.at[pl.ds((wid * sww + t) * W, W)]

        for b in range(2):
            pltpu.async_copy(mslot(b), mbuf.at[b], msem.at[b])
        for t in range(sww):
            b = t % 2
            pltpu.make_async_copy(mslot(t), mbuf.at[b], msem.at[b]).wait()
            pltpu.sync_copy(mbuf.at[b], acc.at[idx_v.at[t]], add=True)
            if t + 2 < sww:
                pltpu.async_copy(mslot(t + 2), mbuf.at[b], msem.at[b])

        plsc.subcore_barrier()
        pltpu.sync_copy(acc.at[pl.ds(r0, stripe)],
                        part_hbm.at[c, pl.ds(r0, stripe)])

    return kern(msgs, ridx, zeros)


def _mlp_body(gs_ref, gr_ref, el_ref, w1s_ref, w1r_ref, w1e_ref, b1_ref,
              w2_ref, b2_ref, o_ref):
    dn_t = (((0,), (1,)), ((), ()))
    x = lax.dot_general(w1s_ref[...], gs_ref[...].astype(jnp.bfloat16), dn_t,
                        preferred_element_type=jnp.float32)
    x += lax.dot_general(w1r_ref[...], gr_ref[...].astype(jnp.bfloat16), dn_t,
                         preferred_element_type=jnp.float32)
    el = el_ref[0]
    x += lax.dot_general(w1e_ref[...], el, (((0,), (0,)), ((), ())),
                         preferred_element_type=jnp.float32)
    x += b1_ref[...]
    hmid = (x * jax.nn.sigmoid(x)).astype(jnp.bfloat16)
    m = lax.dot_general(hmid, w2_ref[...], (((0,), (0,)), ((), ())),
                        preferred_element_type=jnp.float32)
    o_ref[...] = m + b2_ref[...]


def _mlp(gathered, el3, w1s, w1r, w1e, b1c, w2, b2r):
    nbc = gathered.shape[0] // (2 * EB)
    return pl.pallas_call(
        _mlp_body,
        grid=(nbc,),
        in_specs=[
            pl.BlockSpec((EB, HIDDEN), lambda i: (i, 0)),
            pl.BlockSpec((EB, HIDDEN), lambda i: (i + nbc, 0)),
            pl.BlockSpec((1, 1, EB), lambda i: (i, 0, 0)),
            pl.BlockSpec((HIDDEN, HIDDEN), lambda i: (0, 0)),
            pl.BlockSpec((HIDDEN, HIDDEN), lambda i: (0, 0)),
            pl.BlockSpec((1, HIDDEN), lambda i: (0, 0)),
            pl.BlockSpec((HIDDEN, 1), lambda i: (0, 0)),
            pl.BlockSpec((HIDDEN, HIDDEN), lambda i: (0, 0)),
            pl.BlockSpec((1, HIDDEN), lambda i: (0, 0)),
        ],
        out_specs=pl.BlockSpec((EB, HIDDEN), lambda i: (i, 0)),
        out_shape=jax.ShapeDtypeStruct((nbc * EB, HIDDEN), jnp.float32),
    )(gathered, gathered, el3, w1s, w1r, w1e, b1c, w2, b2r)


def _final_body(h_ref, *refs):
    o_ref = refs[-1]
    agg = refs[0][0] + refs[0][1]
    for p_ref in refs[1:-1]:
        agg += p_ref[0] + p_ref[1]
    o_ref[:, :HIDDEN] = h_ref[:, :HIDDEN] + agg
    o_ref[:, HIDDEN:] = h_ref[:, HIDDEN:]


def _finalize(h, parts):
    n, f = h.shape
    rb = 1000
    pspec = pl.BlockSpec((2, rb, HIDDEN), lambda i: (0, i, 0))
    return pl.pallas_call(
        _final_body,
        grid=(n // rb,),
        in_specs=[pl.BlockSpec((rb, f), lambda i: (i, 0))]
        + [pspec] * len(parts),
        out_specs=pl.BlockSpec((rb, f), lambda i: (i, 0)),
        out_shape=jax.ShapeDtypeStruct((n, f), jnp.float32),
    )(h, *parts)


def kernel(h, edge_index, edge_len, W1, b1, W2, b2):
    scalars = jnp.pad(h[:, :HIDDEN], ((0, ACC_ROWS - N_NODES), (0, 0)))
    sender = edge_index[0].astype(jnp.int32)
    receiver = edge_index[1].astype(jnp.int32)
    e = sender.shape[0]
    pad = E_PAD - e
    sender_p = jnp.pad(sender, (0, pad))
    receiver_p = jnp.pad(receiver, (0, pad), constant_values=N_NODES)
    el_p = jnp.pad(edge_len.astype(jnp.float32), (0, pad))

    w1s = W1[:HIDDEN].astype(jnp.bfloat16)
    w1r = W1[HIDDEN:2 * HIDDEN].astype(jnp.bfloat16)
    w1e = W1[2 * HIDDEN:]
    b1c = b1.reshape(HIDDEN, 1)
    w2 = W2.astype(jnp.bfloat16)
    b2r = b2.reshape(1, HIDDEN)

    zeros = jnp.zeros((ACC_ROWS, HIDDEN), jnp.float32)
    parts = []
    off = 0
    for nb in CHUNK_BLOCKS:
        ec = nb * EB
        sl = slice(off, off + ec)
        off += ec
        idx_c = jnp.concatenate(
            [sender_p[sl], receiver_p[sl]]).reshape(2 * ec // W, W)
        gathered = _gather(scalars, idx_c)
        msgs = _mlp(gathered, el_p[sl].reshape(nb, 1, EB),
                    w1s, w1r, w1e, b1c, w2, b2r)
        sww = ec // W // NW
        parts.append(
            _scatter_add(msgs, receiver_p[sl].reshape(NW, sww, W), zeros, sww))
    return _finalize(h, parts)
